# baseline (speedup 1.0000x reference)
import functools

import jax
import jax.numpy as jnp
from jax import lax
from jax.experimental import pallas as pl
from jax.experimental.pallas import tpu as pltpu
from jax.experimental.pallas import tpu_sc as plsc

_N = 10000
_E = 320000
_D = 128
_H = 16
_C = 40

_NC = 2
_NS = 16
_NW = _NC * _NS
_CH = 128
_RPW = 80
_ROWS = _RPW * _NW
_EP = _ROWS * _CH
_NACC = 10240
_ZR = _NACC // _NS
_R0 = 64
_R1 = _RPW * 2 - _R0

_mesh = plsc.VectorSubcoreMesh(core_axis_name="c", subcore_axis_name="s")


@functools.partial(
    pl.kernel,
    out_type=jax.ShapeDtypeStruct((_NC, _NACC), jnp.float32),
    mesh=_mesh,
    compiler_params=pltpu.CompilerParams(use_tc_tiling_on_sc=False),
    scratch_types=[
        pltpu.VMEM((_RPW, _CH), jnp.int32),
        pltpu.VMEM((_CH,), jnp.float32),
        pltpu.VMEM((_ZR,), jnp.float32),
        pltpu.VMEM_SHARED((_NACC,), jnp.float32),
    ],
)
def _deg_kernel(dst_hbm, out_hbm, dst_v, ones_v, zb_v, acc_sh):
    cid = lax.axis_index("c")
    sid = lax.axis_index("s")
    wid = sid * _NC + cid

    def zstep(i, _):
        zb_v[pl.ds(i * 16, 16)] = jnp.zeros((16,), jnp.float32)
        return 0

    lax.fori_loop(0, _ZR // 16, zstep, 0)
    for i in range(_CH // 16):
        ones_v[pl.ds(i * 16, 16)] = jnp.ones((16,), jnp.float32)
    pltpu.sync_copy(zb_v, acc_sh.at[pl.ds(sid * _ZR, _ZR)])
    plsc.subcore_barrier()

    pltpu.sync_copy(dst_hbm.at[pl.ds(wid * _RPW, _RPW)], dst_v)

    def step(j, _):
        pltpu.sync_copy(ones_v, acc_sh.at[dst_v.at[j]], add=True)
        return 0

    lax.fori_loop(0, _RPW, step, 0)
    plsc.subcore_barrier()
    pltpu.sync_copy(acc_sh.at[pl.ds(sid * _ZR, _ZR)],
                    out_hbm.at[cid, pl.ds(sid * _ZR, _ZR)])


@functools.partial(
    pl.kernel,
    out_type=jax.ShapeDtypeStruct((_NC, _NACC, _H), jnp.float32),
    mesh=_mesh,
    compiler_params=pltpu.CompilerParams(use_tc_tiling_on_sc=False),
    scratch_types=[
        pltpu.VMEM((_R1, _CH), jnp.int32),
        pltpu.VMEM((_R1, _CH), jnp.int32),
        pltpu.VMEM((2, _CH, _H), jnp.float32),
        pltpu.VMEM((_ZR, _H), jnp.float32),
        pltpu.VMEM_SHARED((_NACC, _H), jnp.float32),
        pltpu.SemaphoreType.DMA,
    ],
)
def _agg_kernel(hn_hbm, src_hbm, dst_hbm, out_hbm,
                src_v, dst_v, rows_v, zb_v, acc_sh, sem):
    cid = lax.axis_index("c")
    sid = lax.axis_index("s")

    def zstep(i, _):
        zb_v[i] = jnp.zeros((_H,), jnp.float32)
        return 0

    lax.fori_loop(0, _ZR, zstep, 0)
    pltpu.sync_copy(zb_v, acc_sh.at[pl.ds(sid * _ZR, _ZR)])
    plsc.subcore_barrier()

    nrows = lax.select(cid == 0, _R0, _R1)
    base = pl.multiple_of(
        lax.select(cid == 0, sid * _R0, _NS * _R0 + sid * _R1), 8)
    pltpu.sync_copy(src_hbm.at[pl.ds(base, _R1)], src_v)
    pltpu.sync_copy(dst_hbm.at[pl.ds(base, _R1)], dst_v)

    pltpu.async_copy(hn_hbm.at[src_v.at[0]], rows_v.at[0], sem)

    def step(j, _):
        nxt = j + 1

        @pl.when(nxt < nrows)
        def _():
            pltpu.async_copy(hn_hbm.at[src_v.at[nxt]], rows_v.at[nxt % 2], sem)

        pltpu.make_async_copy(hn_hbm.at[src_v.at[j]], rows_v.at[j % 2], sem).wait()
        pltpu.sync_copy(rows_v.at[j % 2], acc_sh.at[dst_v.at[j]], add=True)
        return 0

    lax.fori_loop(0, nrows, step, 0)
    plsc.subcore_barrier()
    pltpu.sync_copy(acc_sh.at[pl.ds(sid * _ZR, _ZR)],
                    out_hbm.at[cid, pl.ds(sid * _ZR, _ZR)])


def _dense1_body(d0, d1, x, w1, hn, dv):
    dinv = lax.rsqrt(d0[...] + d1[...] + 1.0)
    dv[...] = dinv
    hn[...] = jnp.dot(x[...], w1[...], preferred_element_type=jnp.float32) * dinv


def _dense2_body(p0, p1, hn1, dv, b1, w2, hn2):
    s = jnp.maximum(dv[...] * (p0[...] + p1[...] + hn1[...]) + b1[...], 0.0)
    hn2[...] = jnp.dot(s, w2[...], preferred_element_type=jnp.float32) * dv[...]


def _dense3_body(p0, p1, hn2, dv, b2, wo, bo, out):
    s = jnp.maximum(dv[...] * (p0[...] + p1[...] + hn2[...]) + b2[...], 0.0)
    logits = jnp.dot(s, wo[...], preferred_element_type=jnp.float32) + bo[...]
    m = jnp.max(logits, axis=1, keepdims=True)
    lse = jnp.log(jnp.sum(jnp.exp(logits - m), axis=1, keepdims=True)) + m
    out[...] = logits - lse


def kernel(x, edge_index, W1, b1, W2, b2, Wo, bo):
    src = edge_index[0]
    dst = edge_index[1]
    pad = _EP - _E
    srcp = jnp.concatenate([src, jnp.zeros((pad,), jnp.int32)]).reshape(_ROWS, _CH)
    dstp = jnp.concatenate([dst, jnp.full((pad,), _N, jnp.int32)]).reshape(_ROWS, _CH)

    degp = _deg_kernel(dstp)
    d0 = degp[0, :_N].reshape(_N, 1)
    d1 = degp[1, :_N].reshape(_N, 1)

    hn1, dinv = pl.pallas_call(
        _dense1_body,
        out_shape=[jax.ShapeDtypeStruct((_N, _H), jnp.float32),
                   jax.ShapeDtypeStruct((_N, 1), jnp.float32)],
    )(d0, d1, x, W1)

    a1 = _agg_kernel(hn1, srcp, dstp)
    hn2 = pl.pallas_call(
        _dense2_body,
        out_shape=jax.ShapeDtypeStruct((_N, _H), jnp.float32),
    )(a1[0, :_N], a1[1, :_N], hn1, dinv, b1.reshape(1, _H), W2)

    a2 = _agg_kernel(hn2, srcp, dstp)
    out = pl.pallas_call(
        _dense3_body,
        out_shape=jax.ShapeDtypeStruct((_N, _C), jnp.float32),
    )(a2[0, :_N], a2[1, :_N], hn2, dinv, b2.reshape(1, _H), Wo, bo.reshape(1, _C))
    return out


# speedup vs baseline: 1.2560x; 1.2560x over previous
import functools

import jax
import jax.numpy as jnp
from jax import lax
from jax.scipy.linalg import block_diag
from jax.experimental import pallas as pl
from jax.experimental.pallas import tpu as pltpu
from jax.experimental.pallas import tpu_sc as plsc

_N = 10000
_E = 320000
_D = 128
_H = 16
_C = 40

_NC = 2
_NS = 16
_NW = _NC * _NS
_CH = 128
_RPW = 80
_ROWS = _RPW * _NW
_EP = _ROWS * _CH
_NACC = 10240
_ZR = _NACC // _NS
_RBIG = 96
_RSMALL = _RPW * 2 - _RBIG

_mesh = plsc.VectorSubcoreMesh(core_axis_name="c", subcore_axis_name="s")


@functools.partial(
    pl.kernel,
    out_type=jax.ShapeDtypeStruct((_NC, _NACC, _H), jnp.float32),
    mesh=_mesh,
    compiler_params=pltpu.CompilerParams(use_tc_tiling_on_sc=False),
    scratch_types=[
        pltpu.VMEM((_RPW, _CH), jnp.int32),
        pltpu.VMEM((_CH,), jnp.float32),
        pltpu.VMEM((_ZR,), jnp.float32),
        pltpu.VMEM((_ZR, _H), jnp.float32),
        pltpu.VMEM_SHARED((_NACC,), jnp.float32),
    ],
)
def _deg_kernel(dst_hbm, out_hbm, dst_v, ones_v, zb_v, rep_v, acc_sh):
    cid = lax.axis_index("c")
    sid = lax.axis_index("s")
    wid = sid * _NC + cid

    def zstep(i, _):
        zb_v[pl.ds(i * 16, 16)] = jnp.zeros((16,), jnp.float32)
        return 0

    lax.fori_loop(0, _ZR // 16, zstep, 0)
    for i in range(_CH // 16):
        ones_v[pl.ds(i * 16, 16)] = jnp.ones((16,), jnp.float32)
    pltpu.sync_copy(zb_v, acc_sh.at[pl.ds(sid * _ZR, _ZR)])
    plsc.subcore_barrier()

    pltpu.sync_copy(dst_hbm.at[pl.ds(wid * _RPW, _RPW)], dst_v)

    def step(j, _):
        pltpu.sync_copy(ones_v, acc_sh.at[dst_v.at[j]], add=True)
        return 0

    lax.fori_loop(0, _RPW, step, 0)
    plsc.subcore_barrier()
    pltpu.sync_copy(acc_sh.at[pl.ds(sid * _ZR, _ZR)], zb_v)

    def rstep(g, _):
        v = zb_v[pl.ds(g * 16, 16)]
        for j in range(16):
            rep_v[g * 16 + j] = jnp.broadcast_to(v[j], (_H,))
        return 0

    lax.fori_loop(0, _ZR // 16, rstep, 0)
    pltpu.sync_copy(rep_v, out_hbm.at[cid, pl.ds(sid * _ZR, _ZR)])


@functools.partial(
    pl.kernel,
    out_type=jax.ShapeDtypeStruct((_NC, _NACC, _H), jnp.float32),
    mesh=_mesh,
    compiler_params=pltpu.CompilerParams(use_tc_tiling_on_sc=False),
    scratch_types=[
        pltpu.VMEM((_RBIG, _CH), jnp.int32),
        pltpu.VMEM((_RBIG, _CH), jnp.int32),
        pltpu.VMEM((2, _CH, _H), jnp.float32),
        pltpu.VMEM((_ZR, _H), jnp.float32),
        pltpu.VMEM_SHARED((_NACC, _H), jnp.float32),
        pltpu.SemaphoreType.DMA,
    ],
)
def _agg_kernel(hn_hbm, src_hbm, dst_hbm, out_hbm,
                src_v, dst_v, rows_v, zb_v, acc_sh, sem):
    cid = lax.axis_index("c")
    sid = lax.axis_index("s")

    def zstep(i, _):
        zb_v[i] = jnp.zeros((_H,), jnp.float32)
        return 0

    lax.fori_loop(0, _ZR, zstep, 0)
    pltpu.sync_copy(zb_v, acc_sh.at[pl.ds(sid * _ZR, _ZR)])
    plsc.subcore_barrier()

    nrows = lax.select(cid == 0, _RBIG, _RSMALL)
    base = pl.multiple_of(
        lax.select(cid == 0, _NS * _RSMALL + sid * _RBIG, sid * _RSMALL), 8)
    pltpu.sync_copy(src_hbm.at[pl.ds(base, _RBIG)], src_v)
    pltpu.sync_copy(dst_hbm.at[pl.ds(base, _RBIG)], dst_v)

    pltpu.async_copy(hn_hbm.at[src_v.at[0]], rows_v.at[0], sem)

    def step(j, _):
        nxt = j + 1

        @pl.when(nxt < nrows)
        def _():
            pltpu.async_copy(hn_hbm.at[src_v.at[nxt]], rows_v.at[nxt % 2], sem)

        pltpu.make_async_copy(hn_hbm.at[src_v.at[j]], rows_v.at[j % 2], sem).wait()
        pltpu.sync_copy(rows_v.at[j % 2], acc_sh.at[dst_v.at[j]], add=True)
        return 0

    lax.fori_loop(0, nrows, step, 0)
    plsc.subcore_barrier()
    pltpu.sync_copy(acc_sh.at[pl.ds(sid * _ZR, _ZR)],
                    out_hbm.at[cid, pl.ds(sid * _ZR, _ZR)])


def _dense1_body(dgp, xw, w1b, hnp, dvp):
    d = dgp[...]
    dinvp = lax.rsqrt(d[0] + d[1] + 1.0)
    dvp[...] = dinvp
    hnp[...] = jnp.dot(xw[...], w1b[...],
                       preferred_element_type=jnp.float32) * dinvp


def _dense2_body(ap, hnp, dvp, b1p, w2b, hn2p):
    p = ap[...]
    dinvp = dvp[...]
    sp = jnp.maximum((p[0] + p[1] + hnp[...]) * dinvp + b1p[...], 0.0)
    hn2p[...] = jnp.dot(sp, w2b[...], preferred_element_type=jnp.float32) * dinvp


def _dense3_body(ap, hnp, dvp, b2p, wob, bop, outp):
    p = ap[...]
    dinvp = dvp[...]
    sp = jnp.maximum((p[0] + p[1] + hnp[...]) * dinvp + b2p[...], 0.0)
    lp = jnp.dot(sp, wob[...], preferred_element_type=jnp.float32) + bop[...]
    outs = []
    for k in range(8):
        lk = lp[:, 40 * k:40 * (k + 1)]
        m = jnp.max(lk, axis=1, keepdims=True)
        lse = jnp.log(jnp.sum(jnp.exp(lk - m), axis=1, keepdims=True)) + m
        outs.append(lk - lse)
    outp[...] = jnp.concatenate(outs, axis=1)


def kernel(x, edge_index, W1, b1, W2, b2, Wo, bo):
    src = edge_index[0]
    dst = edge_index[1]
    pad = _EP - _E
    srcp = jnp.concatenate([src, jnp.zeros((pad,), jnp.int32)]).reshape(_ROWS, _CH)
    dstp = jnp.concatenate([dst, jnp.full((pad,), _N, jnp.int32)]).reshape(_ROWS, _CH)

    degrep = _deg_kernel(dstp)

    _P8 = _NACC // 8
    xw = jnp.reshape(jnp.pad(x, ((0, _NACC - _N), (0, 0))), (_P8, 8 * _D))
    w1b = block_diag(*([W1] * 8))
    hn1p, dinvp = pl.pallas_call(
        _dense1_body,
        out_shape=[jax.ShapeDtypeStruct((_P8, 128), jnp.float32),
                   jax.ShapeDtypeStruct((_P8, 128), jnp.float32)],
    )(jnp.reshape(degrep, (_NC, _P8, 128)), xw, w1b)

    w2b = block_diag(*([W2] * 8))
    b1p = jnp.tile(b1, 8).reshape(1, 128)

    a1 = _agg_kernel(jnp.reshape(hn1p, (_NACC, _H)), srcp, dstp)
    hn2p = pl.pallas_call(
        _dense2_body,
        out_shape=jax.ShapeDtypeStruct((_P8, 128), jnp.float32),
    )(jnp.reshape(a1, (_NC, _P8, 128)), hn1p, dinvp, b1p, w2b)

    wob = block_diag(*([Wo] * 8))
    b2p = jnp.tile(b2, 8).reshape(1, 128)
    bop = jnp.tile(bo, 8).reshape(1, 8 * _C)

    a2 = _agg_kernel(jnp.reshape(hn2p, (_NACC, _H)), srcp, dstp)
    outp = pl.pallas_call(
        _dense3_body,
        out_shape=jax.ShapeDtypeStruct((_P8, 8 * _C), jnp.float32),
    )(jnp.reshape(a2, (_NC, _P8, 128)), hn2p, dinvp, b2p, wob, bop)
    return jnp.reshape(outp, (_NACC, _C))[:_N]


# speedup vs baseline: 1.3552x; 1.0790x over previous
import functools

import jax
import jax.numpy as jnp
from jax import lax
from jax.scipy.linalg import block_diag
from jax.experimental import pallas as pl
from jax.experimental.pallas import tpu as pltpu
from jax.experimental.pallas import tpu_sc as plsc

_N = 10000
_E = 320000
_D = 128
_H = 16
_C = 40

_NC = 2
_NS = 16
_NW = _NC * _NS
_CH = 128
_RPW = 80
_ROWS = _RPW * _NW
_EP = _ROWS * _CH
_NACC = 10240
_ZR = _NACC // _NS
_RBIG = 80
_RSMALL = _RPW * 2 - _RBIG

_mesh = plsc.VectorSubcoreMesh(core_axis_name="c", subcore_axis_name="s")


@functools.partial(
    pl.kernel,
    out_type=jax.ShapeDtypeStruct((_NC, _NACC, _H), jnp.float32),
    mesh=_mesh,
    compiler_params=pltpu.CompilerParams(use_tc_tiling_on_sc=False),
    scratch_types=[
        pltpu.VMEM((_RPW, _CH), jnp.int32),
        pltpu.VMEM((_CH,), jnp.float32),
        pltpu.VMEM((_ZR,), jnp.float32),
        pltpu.VMEM((_ZR, _H), jnp.float32),
        pltpu.VMEM_SHARED((_NACC,), jnp.float32),
    ],
)
def _deg_kernel(dst_hbm, out_hbm, dst_v, ones_v, zb_v, rep_v, acc_sh):
    cid = lax.axis_index("c")
    sid = lax.axis_index("s")
    wid = sid * _NC + cid

    def zstep(i, _):
        zb_v[pl.ds(i * 16, 16)] = jnp.zeros((16,), jnp.float32)
        return 0

    lax.fori_loop(0, _ZR // 16, zstep, 0)
    for i in range(_CH // 16):
        ones_v[pl.ds(i * 16, 16)] = jnp.ones((16,), jnp.float32)
    pltpu.sync_copy(zb_v, acc_sh.at[pl.ds(sid * _ZR, _ZR)])
    plsc.subcore_barrier()

    pltpu.sync_copy(dst_hbm.at[pl.ds(wid * _RPW, _RPW)], dst_v)

    def step(j, _):
        pltpu.sync_copy(ones_v, acc_sh.at[dst_v.at[j]], add=True)
        return 0

    lax.fori_loop(0, _RPW, step, 0)
    plsc.subcore_barrier()
    pltpu.sync_copy(acc_sh.at[pl.ds(sid * _ZR, _ZR)], zb_v)

    def rstep(g, _):
        v = zb_v[pl.ds(g * 16, 16)]
        for j in range(16):
            rep_v[g * 16 + j] = jnp.broadcast_to(v[j], (_H,))
        return 0

    lax.fori_loop(0, _ZR // 16, rstep, 0)
    pltpu.sync_copy(rep_v, out_hbm.at[cid, pl.ds(sid * _ZR, _ZR)])


@functools.partial(
    pl.kernel,
    out_type=jax.ShapeDtypeStruct((_NC, _NACC, _H), jnp.float32),
    mesh=_mesh,
    compiler_params=pltpu.CompilerParams(use_tc_tiling_on_sc=False),
    scratch_types=[
        pltpu.VMEM((_RBIG, _CH), jnp.int32),
        pltpu.VMEM((_RBIG, _CH), jnp.int32),
        pltpu.VMEM((2, _CH, _H), jnp.float32),
        pltpu.VMEM((_ZR, _H), jnp.float32),
        pltpu.VMEM_SHARED((_NACC, _H), jnp.float32),
        pltpu.SemaphoreType.DMA,
    ],
)
def _agg_kernel(hn_hbm, src_hbm, dst_hbm, out_hbm,
                src_v, dst_v, rows_v, zb_v, acc_sh, sem):
    cid = lax.axis_index("c")
    sid = lax.axis_index("s")

    def zstep(i, _):
        zb_v[i] = jnp.zeros((_H,), jnp.float32)
        return 0

    lax.fori_loop(0, _ZR, zstep, 0)
    pltpu.sync_copy(zb_v, acc_sh.at[pl.ds(sid * _ZR, _ZR)])
    plsc.subcore_barrier()

    nrows = lax.select(cid == 0, _RBIG, _RSMALL)
    base = pl.multiple_of(
        lax.select(cid == 0, _NS * _RSMALL + sid * _RBIG, sid * _RSMALL), 8)
    pltpu.sync_copy(src_hbm.at[pl.ds(base, _RBIG)], src_v)
    pltpu.sync_copy(dst_hbm.at[pl.ds(base, _RBIG)], dst_v)

    pltpu.async_copy(hn_hbm.at[src_v.at[0]], rows_v.at[0], sem)

    def step(j, _):
        nxt = j + 1

        @pl.when(nxt < nrows)
        def _():
            pltpu.async_copy(hn_hbm.at[src_v.at[nxt]], rows_v.at[nxt % 2], sem)

        pltpu.make_async_copy(hn_hbm.at[src_v.at[j]], rows_v.at[j % 2], sem).wait()
        pltpu.sync_copy(rows_v.at[j % 2], acc_sh.at[dst_v.at[j]], add=True)
        return 0

    lax.fori_loop(0, nrows, step, 0)
    plsc.subcore_barrier()
    pltpu.sync_copy(acc_sh.at[pl.ds(sid * _ZR, _ZR)],
                    out_hbm.at[cid, pl.ds(sid * _ZR, _ZR)])


def _dense1_body(dgp, xw, w1b, hnp, dvp):
    d = dgp[...]
    dinvp = lax.rsqrt(d[0] + d[1] + 1.0)
    dvp[...] = dinvp
    hnp[...] = jnp.dot(xw[...], w1b[...],
                       preferred_element_type=jnp.float32) * dinvp


def _dense2_body(ap, hnp, dvp, b1p, w2b, hn2p):
    p = ap[...]
    dinvp = dvp[...]
    sp = jnp.maximum((p[0] + p[1] + hnp[...]) * dinvp + b1p[...], 0.0)
    hn2p[...] = jnp.dot(sp, w2b[...], preferred_element_type=jnp.float32) * dinvp


def _dense3_body(ap, hnp, dvp, b2p, wob, bop, outp):
    p = ap[...]
    dinvp = dvp[...]
    sp = jnp.maximum((p[0] + p[1] + hnp[...]) * dinvp + b2p[...], 0.0)
    lp = jnp.dot(sp, wob[...], preferred_element_type=jnp.float32) + bop[...]
    outs = []
    for k in range(8):
        lk = lp[:, 40 * k:40 * (k + 1)]
        m = jnp.max(lk, axis=1, keepdims=True)
        lse = jnp.log(jnp.sum(jnp.exp(lk - m), axis=1, keepdims=True)) + m
        outs.append(lk - lse)
    outp[...] = jnp.concatenate(outs, axis=1)


def kernel(x, edge_index, W1, b1, W2, b2, Wo, bo):
    src = edge_index[0]
    dst = edge_index[1]
    pad = _EP - _E
    srcp = jnp.concatenate([src, jnp.zeros((pad,), jnp.int32)]).reshape(_ROWS, _CH)
    dump = _N + (jnp.arange(pad, dtype=jnp.int32) % (_NACC - _N))
    dstp = jnp.concatenate([dst, dump]).reshape(_ROWS, _CH)

    degrep = _deg_kernel(dstp)

    _P8 = _NACC // 8
    xw = jnp.reshape(jnp.pad(x, ((0, _NACC - _N), (0, 0))), (_P8, 8 * _D))
    w1b = block_diag(*([W1] * 8))
    hn1p, dinvp = pl.pallas_call(
        _dense1_body,
        out_shape=[jax.ShapeDtypeStruct((_P8, 128), jnp.float32),
                   jax.ShapeDtypeStruct((_P8, 128), jnp.float32)],
    )(jnp.reshape(degrep, (_NC, _P8, 128)), xw, w1b)

    w2b = block_diag(*([W2] * 8))
    b1p = jnp.tile(b1, 8).reshape(1, 128)

    a1 = _agg_kernel(jnp.reshape(hn1p, (_NACC, _H)), srcp, dstp)
    hn2p = pl.pallas_call(
        _dense2_body,
        out_shape=jax.ShapeDtypeStruct((_P8, 128), jnp.float32),
    )(jnp.reshape(a1, (_NC, _P8, 128)), hn1p, dinvp, b1p, w2b)

    wob = block_diag(*([Wo] * 8))
    b2p = jnp.tile(b2, 8).reshape(1, 128)
    bop = jnp.tile(bo, 8).reshape(1, 8 * _C)

    a2 = _agg_kernel(jnp.reshape(hn2p, (_NACC, _H)), srcp, dstp)
    outp = pl.pallas_call(
        _dense3_body,
        out_shape=jax.ShapeDtypeStruct((_P8, 8 * _C), jnp.float32),
    )(jnp.reshape(a2, (_NC, _P8, 128)), hn2p, dinvp, b2p, wob, bop)
    return jnp.reshape(outp, (_NACC, _C))[:_N]


# speedup vs baseline: 2.4097x; 1.7781x over previous
import functools

import jax
import jax.numpy as jnp
from jax import lax
from jax.scipy.linalg import block_diag
from jax.experimental import pallas as pl
from jax.experimental.pallas import tpu as pltpu
from jax.experimental.pallas import tpu_sc as plsc

_N = 10000
_E = 320000
_D = 128
_H = 16
_C = 40

_NC = 2
_NS = 16
_NW = _NC * _NS
_CH = 128
_RPW = 80
_ROWS = _RPW * _NW
_EP = _ROWS * _CH
_NACC = 10240
_ZR = _NACC // _NS
_RBIG = 80
_RSMALL = _RPW * 2 - _RBIG

_mesh = plsc.VectorSubcoreMesh(core_axis_name="c", subcore_axis_name="s")


@functools.partial(
    pl.kernel,
    out_type=jax.ShapeDtypeStruct((_NC, _NACC, _H), jnp.float32),
    mesh=_mesh,
    compiler_params=pltpu.CompilerParams(use_tc_tiling_on_sc=False),
    scratch_types=[
        pltpu.VMEM((_RPW, _CH), jnp.int32),
        pltpu.VMEM((_CH,), jnp.float32),
        pltpu.VMEM((_ZR,), jnp.float32),
        pltpu.VMEM((_ZR, _H), jnp.float32),
        pltpu.VMEM_SHARED((_NACC,), jnp.float32),
    ],
)
def _deg_kernel(dst_hbm, out_hbm, dst_v, ones_v, zb_v, rep_v, acc_sh):
    cid = lax.axis_index("c")
    sid = lax.axis_index("s")
    wid = sid * _NC + cid

    def zstep(i, _):
        zb_v[pl.ds(i * 16, 16)] = jnp.zeros((16,), jnp.float32)
        return 0

    lax.fori_loop(0, _ZR // 16, zstep, 0)
    for i in range(_CH // 16):
        ones_v[pl.ds(i * 16, 16)] = jnp.ones((16,), jnp.float32)
    pltpu.sync_copy(zb_v, acc_sh.at[pl.ds(sid * _ZR, _ZR)])
    plsc.subcore_barrier()

    pltpu.sync_copy(dst_hbm.at[pl.ds(wid * _RPW, _RPW)], dst_v)

    def step(j, _):
        pltpu.sync_copy(ones_v, acc_sh.at[dst_v.at[j]], add=True)
        return 0

    lax.fori_loop(0, _RPW, step, 0)
    plsc.subcore_barrier()
    pltpu.sync_copy(acc_sh.at[pl.ds(sid * _ZR, _ZR)], zb_v)

    def rstep(g, _):
        v = zb_v[pl.ds(g * 16, 16)]
        for j in range(16):
            rep_v[g * 16 + j] = jnp.broadcast_to(v[j], (_H,))
        return 0

    lax.fori_loop(0, _ZR // 16, rstep, 0)
    pltpu.sync_copy(rep_v, out_hbm.at[cid, pl.ds(sid * _ZR, _ZR)])


@functools.partial(
    pl.kernel,
    out_type=jax.ShapeDtypeStruct((_NC, _NACC, _H), jnp.float32),
    mesh=_mesh,
    compiler_params=pltpu.CompilerParams(use_tc_tiling_on_sc=False),
    scratch_types=[
        pltpu.VMEM((_RBIG, _CH), jnp.int32),
        pltpu.VMEM((_RBIG, _CH), jnp.int32),
        pltpu.VMEM((8, _CH, _H), jnp.float32),
        pltpu.VMEM((_ZR, _H), jnp.float32),
        pltpu.VMEM_SHARED((_NACC, _H), jnp.float32),
        pltpu.SemaphoreType.DMA,
        pltpu.SemaphoreType.DMA,
        pltpu.SemaphoreType.DMA,
    ],
)
def _agg_kernel(hn_hbm, src_hbm, dst_hbm, out_hbm,
                src_v, dst_v, rows_v, zb_v, acc_sh, sem_g, sem_s0, sem_s1):
    cid = lax.axis_index("c")
    sid = lax.axis_index("s")

    def zstep(i, _):
        zb_v[i] = jnp.zeros((_H,), jnp.float32)
        return 0

    lax.fori_loop(0, _ZR, zstep, 0)
    pltpu.sync_copy(zb_v, acc_sh.at[pl.ds(sid * _ZR, _ZR)])
    plsc.subcore_barrier()

    nrows = lax.select(cid == 0, _RBIG, _RSMALL)
    base = pl.multiple_of(
        lax.select(cid == 0, _NS * _RSMALL + sid * _RBIG, sid * _RSMALL), 8)
    pltpu.sync_copy(src_hbm.at[pl.ds(base, _RBIG)], src_v)
    pltpu.sync_copy(dst_hbm.at[pl.ds(base, _RBIG)], dst_v)

    nblk = nrows // 8

    for t in range(4):
        pltpu.async_copy(hn_hbm.at[src_v.at[t]], rows_v.at[t], sem_g)

    def blk(k, _):
        j0 = k * 8

        @pl.when(k >= 1)
        def _():
            for t in range(4):
                pltpu.make_async_copy(rows_v.at[4 + t],
                                      acc_sh.at[dst_v.at[j0 - 4 + t]],
                                      sem_s1).wait()

        for t in range(4):
            pltpu.async_copy(hn_hbm.at[src_v.at[j0 + 4 + t]],
                             rows_v.at[4 + t], sem_g)
        for t in range(4):
            pltpu.make_async_copy(hn_hbm.at[src_v.at[j0 + t]],
                                  rows_v.at[t], sem_g).wait()
            pltpu.async_copy(rows_v.at[t], acc_sh.at[dst_v.at[j0 + t]],
                             sem_s0, add=True)
        for t in range(4):
            pltpu.make_async_copy(rows_v.at[t], acc_sh.at[dst_v.at[j0 + t]],
                                  sem_s0).wait()
        for t in range(4):
            pltpu.make_async_copy(hn_hbm.at[src_v.at[j0 + 4 + t]],
                                  rows_v.at[4 + t], sem_g).wait()
            pltpu.async_copy(rows_v.at[4 + t], acc_sh.at[dst_v.at[j0 + 4 + t]],
                             sem_s1, add=True)

        @pl.when(k + 1 < nblk)
        def _():
            for t in range(4):
                pltpu.async_copy(hn_hbm.at[src_v.at[j0 + 8 + t]],
                                 rows_v.at[t], sem_g)
        return 0

    lax.fori_loop(0, nblk, blk, 0)
    for t in range(4):
        pltpu.make_async_copy(rows_v.at[4 + t], acc_sh.at[dst_v.at[0]],
                              sem_s1).wait()
    plsc.subcore_barrier()
    pltpu.sync_copy(acc_sh.at[pl.ds(sid * _ZR, _ZR)],
                    out_hbm.at[cid, pl.ds(sid * _ZR, _ZR)])


def _dense1_body(dgp, xw, w1b, hnp, dvp):
    d = dgp[...]
    dinvp = lax.rsqrt(d[0] + d[1] + 1.0)
    dvp[...] = dinvp
    hnp[...] = jnp.dot(xw[...], w1b[...],
                       preferred_element_type=jnp.float32) * dinvp


def _dense2_body(ap, hnp, dvp, b1p, w2b, hn2p):
    p = ap[...]
    dinvp = dvp[...]
    sp = jnp.maximum((p[0] + p[1] + hnp[...]) * dinvp + b1p[...], 0.0)
    hn2p[...] = jnp.dot(sp, w2b[...], preferred_element_type=jnp.float32) * dinvp


def _dense3_body(ap, hnp, dvp, b2p, wob, bop, outp):
    p = ap[...]
    dinvp = dvp[...]
    sp = jnp.maximum((p[0] + p[1] + hnp[...]) * dinvp + b2p[...], 0.0)
    lp = jnp.dot(sp, wob[...], preferred_element_type=jnp.float32) + bop[...]
    outs = []
    for k in range(8):
        lk = lp[:, 40 * k:40 * (k + 1)]
        m = jnp.max(lk, axis=1, keepdims=True)
        lse = jnp.log(jnp.sum(jnp.exp(lk - m), axis=1, keepdims=True)) + m
        outs.append(lk - lse)
    outp[...] = jnp.concatenate(outs, axis=1)


def kernel(x, edge_index, W1, b1, W2, b2, Wo, bo):
    src = edge_index[0]
    dst = edge_index[1]
    pad = _EP - _E
    fill = jnp.arange(pad, dtype=jnp.int32)
    srcp = jnp.concatenate([src, fill % _N]).reshape(_ROWS, _CH)
    dump = _N + (fill % (_NACC - _N))
    dstp = jnp.concatenate([dst, dump]).reshape(_ROWS, _CH)

    degrep = _deg_kernel(dstp)

    _P8 = _NACC // 8
    xw = jnp.reshape(jnp.pad(x, ((0, _NACC - _N), (0, 0))), (_P8, 8 * _D))
    w1b = block_diag(*([W1] * 8))
    hn1p, dinvp = pl.pallas_call(
        _dense1_body,
        out_shape=[jax.ShapeDtypeStruct((_P8, 128), jnp.float32),
                   jax.ShapeDtypeStruct((_P8, 128), jnp.float32)],
    )(jnp.reshape(degrep, (_NC, _P8, 128)), xw, w1b)

    w2b = block_diag(*([W2] * 8))
    b1p = jnp.tile(b1, 8).reshape(1, 128)

    a1 = _agg_kernel(jnp.reshape(hn1p, (_NACC, _H)), srcp, dstp)
    hn2p = pl.pallas_call(
        _dense2_body,
        out_shape=jax.ShapeDtypeStruct((_P8, 128), jnp.float32),
    )(jnp.reshape(a1, (_NC, _P8, 128)), hn1p, dinvp, b1p, w2b)

    wob = block_diag(*([Wo] * 8))
    b2p = jnp.tile(b2, 8).reshape(1, 128)
    bop = jnp.tile(bo, 8).reshape(1, 8 * _C)

    a2 = _agg_kernel(jnp.reshape(hn2p, (_NACC, _H)), srcp, dstp)
    outp = pl.pallas_call(
        _dense3_body,
        out_shape=jax.ShapeDtypeStruct((_P8, 8 * _C), jnp.float32),
    )(jnp.reshape(a2, (_NC, _P8, 128)), hn2p, dinvp, b2p, wob, bop)
    return jnp.reshape(outp, (_NACC, _C))[:_N]


# speedup vs baseline: 2.5699x; 1.0665x over previous
import functools

import jax
import jax.numpy as jnp
from jax import lax
from jax.scipy.linalg import block_diag
from jax.experimental import pallas as pl
from jax.experimental.pallas import tpu as pltpu
from jax.experimental.pallas import tpu_sc as plsc

_N = 10000
_E = 320000
_D = 128
_H = 16
_C = 40

_NC = 2
_NS = 16
_NW = _NC * _NS
_CH = 128
_RPW = 80
_ROWS = _RPW * _NW
_EP = _ROWS * _CH
_NACC = 10240
_ZR = _NACC // _NS
_RBIG = 80
_RSMALL = _RPW * 2 - _RBIG

_mesh = plsc.VectorSubcoreMesh(core_axis_name="c", subcore_axis_name="s")


@functools.partial(
    pl.kernel,
    out_type=jax.ShapeDtypeStruct((_NC, _NACC, _H), jnp.float32),
    mesh=_mesh,
    compiler_params=pltpu.CompilerParams(use_tc_tiling_on_sc=False),
    scratch_types=[
        pltpu.VMEM((_RPW, _CH), jnp.int32),
        pltpu.VMEM((_CH,), jnp.float32),
        pltpu.VMEM((_ZR,), jnp.float32),
        pltpu.VMEM((_ZR, _H), jnp.float32),
        pltpu.VMEM_SHARED((_NACC,), jnp.float32),
        pltpu.SemaphoreType.DMA,
    ],
)
def _deg_kernel(dst_hbm, out_hbm, dst_v, ones_v, zb_v, rep_v, acc_sh, sem):
    cid = lax.axis_index("c")
    sid = lax.axis_index("s")
    wid = sid * _NC + cid

    def zstep(i, _):
        zb_v[pl.ds(i * 16, 16)] = jnp.zeros((16,), jnp.float32)
        return 0

    lax.fori_loop(0, _ZR // 16, zstep, 0)
    for i in range(_CH // 16):
        ones_v[pl.ds(i * 16, 16)] = jnp.ones((16,), jnp.float32)
    pltpu.sync_copy(zb_v, acc_sh.at[pl.ds(sid * _ZR, _ZR)])
    plsc.subcore_barrier()

    pltpu.sync_copy(dst_hbm.at[pl.ds(wid * _RPW, _RPW)], dst_v)

    for t in range(8):
        pltpu.async_copy(ones_v, acc_sh.at[dst_v.at[t]], sem, add=True)

    def step(j, _):
        @pl.when(j + 8 < _RPW)
        def _():
            pltpu.async_copy(ones_v, acc_sh.at[dst_v.at[j + 8]], sem, add=True)

        pltpu.make_async_copy(ones_v, acc_sh.at[dst_v.at[0]], sem).wait()
        return 0

    lax.fori_loop(0, _RPW, step, 0)
    plsc.subcore_barrier()
    pltpu.sync_copy(acc_sh.at[pl.ds(sid * _ZR, _ZR)], zb_v)

    def rstep(g, _):
        v = zb_v[pl.ds(g * 16, 16)]
        for j in range(16):
            rep_v[g * 16 + j] = jnp.broadcast_to(v[j], (_H,))
        return 0

    lax.fori_loop(0, _ZR // 16, rstep, 0)
    pltpu.sync_copy(rep_v, out_hbm.at[cid, pl.ds(sid * _ZR, _ZR)])


@functools.partial(
    pl.kernel,
    out_type=jax.ShapeDtypeStruct((_NC, _NACC, _H), jnp.float32),
    mesh=_mesh,
    compiler_params=pltpu.CompilerParams(use_tc_tiling_on_sc=False),
    scratch_types=[
        pltpu.VMEM((_RBIG, _CH), jnp.int32),
        pltpu.VMEM((_RBIG, _CH), jnp.int32),
        pltpu.VMEM((8, _CH, _H), jnp.float32),
        pltpu.VMEM((_ZR, _H), jnp.float32),
        pltpu.VMEM_SHARED((_NACC, _H), jnp.float32),
        pltpu.SemaphoreType.DMA,
        pltpu.SemaphoreType.DMA,
        pltpu.SemaphoreType.DMA,
    ],
)
def _agg_kernel(hn_hbm, src_hbm, dst_hbm, out_hbm,
                src_v, dst_v, rows_v, zb_v, acc_sh, sem_g, sem_s0, sem_s1):
    cid = lax.axis_index("c")
    sid = lax.axis_index("s")

    def zstep(i, _):
        zb_v[i] = jnp.zeros((_H,), jnp.float32)
        return 0

    lax.fori_loop(0, _ZR, zstep, 0)
    pltpu.sync_copy(zb_v, acc_sh.at[pl.ds(sid * _ZR, _ZR)])
    plsc.subcore_barrier()

    nrows = lax.select(cid == 0, _RBIG, _RSMALL)
    base = pl.multiple_of(
        lax.select(cid == 0, _NS * _RSMALL + sid * _RBIG, sid * _RSMALL), 8)
    pltpu.sync_copy(src_hbm.at[pl.ds(base, _RBIG)], src_v)
    pltpu.sync_copy(dst_hbm.at[pl.ds(base, _RBIG)], dst_v)

    nblk = nrows // 8

    for t in range(4):
        pltpu.async_copy(hn_hbm.at[src_v.at[t]], rows_v.at[t], sem_g)

    def blk(k, _):
        j0 = k * 8

        @pl.when(k >= 1)
        def _():
            for t in range(4):
                pltpu.make_async_copy(rows_v.at[4 + t],
                                      acc_sh.at[dst_v.at[j0 - 4 + t]],
                                      sem_s1).wait()

        for t in range(4):
            pltpu.async_copy(hn_hbm.at[src_v.at[j0 + 4 + t]],
                             rows_v.at[4 + t], sem_g)
        for t in range(4):
            pltpu.make_async_copy(hn_hbm.at[src_v.at[j0 + t]],
                                  rows_v.at[t], sem_g).wait()
            pltpu.async_copy(rows_v.at[t], acc_sh.at[dst_v.at[j0 + t]],
                             sem_s0, add=True)
        for t in range(4):
            pltpu.make_async_copy(rows_v.at[t], acc_sh.at[dst_v.at[j0 + t]],
                                  sem_s0).wait()
        for t in range(4):
            pltpu.make_async_copy(hn_hbm.at[src_v.at[j0 + 4 + t]],
                                  rows_v.at[4 + t], sem_g).wait()
            pltpu.async_copy(rows_v.at[4 + t], acc_sh.at[dst_v.at[j0 + 4 + t]],
                             sem_s1, add=True)

        @pl.when(k + 1 < nblk)
        def _():
            for t in range(4):
                pltpu.async_copy(hn_hbm.at[src_v.at[j0 + 8 + t]],
                                 rows_v.at[t], sem_g)
        return 0

    lax.fori_loop(0, nblk, blk, 0)
    for t in range(4):
        pltpu.make_async_copy(rows_v.at[4 + t], acc_sh.at[dst_v.at[0]],
                              sem_s1).wait()
    plsc.subcore_barrier()
    pltpu.sync_copy(acc_sh.at[pl.ds(sid * _ZR, _ZR)],
                    out_hbm.at[cid, pl.ds(sid * _ZR, _ZR)])


def _dense1_body(dgp, xw, w1b, hnp, dvp):
    d = dgp[...]
    dinvp = lax.rsqrt(d[0] + d[1] + 1.0)
    dvp[...] = dinvp
    hnp[...] = jnp.dot(xw[...], w1b[...],
                       preferred_element_type=jnp.float32) * dinvp


def _dense2_body(ap, hnp, dvp, b1p, w2b, hn2p):
    p = ap[...]
    dinvp = dvp[...]
    sp = jnp.maximum((p[0] + p[1] + hnp[...]) * dinvp + b1p[...], 0.0)
    hn2p[...] = jnp.dot(sp, w2b[...], preferred_element_type=jnp.float32) * dinvp


def _dense3_body(ap, hnp, dvp, b2p, wob, bop, outp):
    p = ap[...]
    dinvp = dvp[...]
    sp = jnp.maximum((p[0] + p[1] + hnp[...]) * dinvp + b2p[...], 0.0)
    lp = jnp.dot(sp, wob[...], preferred_element_type=jnp.float32) + bop[...]
    outs = []
    for k in range(8):
        lk = lp[:, 40 * k:40 * (k + 1)]
        m = jnp.max(lk, axis=1, keepdims=True)
        lse = jnp.log(jnp.sum(jnp.exp(lk - m), axis=1, keepdims=True)) + m
        outs.append(lk - lse)
    outp[...] = jnp.concatenate(outs, axis=1)[:_N // 8, :]


def kernel(x, edge_index, W1, b1, W2, b2, Wo, bo):
    src = edge_index[0]
    dst = edge_index[1]
    pad = _EP - _E
    fill = jnp.arange(pad, dtype=jnp.int32)
    srcp = jnp.concatenate([src, fill % _N]).reshape(_ROWS, _CH)
    dump = _N + (fill % (_NACC - _N))
    dstp = jnp.concatenate([dst, dump]).reshape(_ROWS, _CH)

    degrep = _deg_kernel(dstp)

    _P8 = _NACC // 8
    xw = jnp.reshape(jnp.pad(x, ((0, _NACC - _N), (0, 0))), (_P8, 8 * _D))
    w1b = block_diag(*([W1] * 8))
    hn1p, dinvp = pl.pallas_call(
        _dense1_body,
        out_shape=[jax.ShapeDtypeStruct((_P8, 128), jnp.float32),
                   jax.ShapeDtypeStruct((_P8, 128), jnp.float32)],
    )(jnp.reshape(degrep, (_NC, _P8, 128)), xw, w1b)

    w2b = block_diag(*([W2] * 8))
    b1p = jnp.tile(b1, 8).reshape(1, 128)

    a1 = _agg_kernel(jnp.reshape(hn1p, (_NACC, _H)), srcp, dstp)
    hn2p = pl.pallas_call(
        _dense2_body,
        out_shape=jax.ShapeDtypeStruct((_P8, 128), jnp.float32),
    )(jnp.reshape(a1, (_NC, _P8, 128)), hn1p, dinvp, b1p, w2b)

    wob = block_diag(*([Wo] * 8))
    b2p = jnp.tile(b2, 8).reshape(1, 128)
    bop = jnp.tile(bo, 8).reshape(1, 8 * _C)

    a2 = _agg_kernel(jnp.reshape(hn2p, (_NACC, _H)), srcp, dstp)
    outp = pl.pallas_call(
        _dense3_body,
        out_shape=jax.ShapeDtypeStruct((_N // 8, 8 * _C), jnp.float32),
    )(jnp.reshape(a2, (_NC, _P8, 128)), hn2p, dinvp, b2p, wob, bop)
    return jnp.reshape(outp, (_N, _C))


# speedup vs baseline: 2.7173x; 1.0573x over previous
import functools

import jax
import jax.numpy as jnp
from jax import lax
from jax.scipy.linalg import block_diag
from jax.experimental import pallas as pl
from jax.experimental.pallas import tpu as pltpu
from jax.experimental.pallas import tpu_sc as plsc

_N = 10000
_E = 320000
_D = 128
_H = 16
_C = 40

_NC = 2
_NS = 16
_NW = _NC * _NS
_CH = 128
_RPW = 80
_ROWS = _RPW * _NW
_EP = _ROWS * _CH
_NACC = 10240
_ZR = _NACC // _NS
_RBIG = 80
_RSMALL = _RPW * 2 - _RBIG

_mesh = plsc.VectorSubcoreMesh(core_axis_name="c", subcore_axis_name="s")


@functools.partial(
    pl.kernel,
    out_type=jax.ShapeDtypeStruct((_NC, _NACC, _H), jnp.float32),
    mesh=_mesh,
    compiler_params=pltpu.CompilerParams(use_tc_tiling_on_sc=False),
    scratch_types=[
        pltpu.VMEM((_RPW, _CH), jnp.int32),
        pltpu.VMEM((_CH,), jnp.float32),
        pltpu.VMEM((_ZR,), jnp.float32),
        pltpu.VMEM((_ZR, _H), jnp.float32),
        pltpu.VMEM_SHARED((_NACC,), jnp.float32),
        pltpu.SemaphoreType.DMA,
    ],
)
def _deg_kernel(dst_hbm, out_hbm, dst_v, ones_v, zb_v, rep_v, acc_sh, sem):
    cid = lax.axis_index("c")
    sid = lax.axis_index("s")
    wid = sid * _NC + cid

    def zstep(i, _):
        zb_v[pl.ds(i * 16, 16)] = jnp.zeros((16,), jnp.float32)
        return 0

    lax.fori_loop(0, _ZR // 16, zstep, 0)
    for i in range(_CH // 16):
        ones_v[pl.ds(i * 16, 16)] = jnp.ones((16,), jnp.float32)
    pltpu.sync_copy(zb_v, acc_sh.at[pl.ds(sid * _ZR, _ZR)])
    plsc.subcore_barrier()

    pltpu.sync_copy(dst_hbm.at[pl.ds(wid * _RPW, _RPW)], dst_v)

    for t in range(8):
        pltpu.async_copy(ones_v, acc_sh.at[dst_v.at[t]], sem, add=True)

    def step(j, _):
        @pl.when(j + 8 < _RPW)
        def _():
            pltpu.async_copy(ones_v, acc_sh.at[dst_v.at[j + 8]], sem, add=True)

        pltpu.make_async_copy(ones_v, acc_sh.at[dst_v.at[0]], sem).wait()
        return 0

    lax.fori_loop(0, _RPW, step, 0)
    plsc.subcore_barrier()
    pltpu.sync_copy(acc_sh.at[pl.ds(sid * _ZR, _ZR)], zb_v)

    def rstep(g, _):
        v = zb_v[pl.ds(g * 16, 16)]
        for j in range(16):
            rep_v[g * 16 + j] = jnp.broadcast_to(v[j], (_H,))
        return 0

    lax.fori_loop(0, _ZR // 16, rstep, 0)
    pltpu.sync_copy(rep_v, out_hbm.at[cid, pl.ds(sid * _ZR, _ZR)])


@functools.partial(
    pl.kernel,
    out_type=jax.ShapeDtypeStruct((_NC, _NACC, _H), jnp.float32),
    mesh=_mesh,
    compiler_params=pltpu.CompilerParams(use_tc_tiling_on_sc=False),
    scratch_types=[
        pltpu.VMEM((_RBIG, _CH), jnp.int32),
        pltpu.VMEM((_RBIG, _CH), jnp.int32),
        pltpu.VMEM((16, _CH, _H), jnp.float32),
        pltpu.VMEM((_ZR, _H), jnp.float32),
        pltpu.VMEM_SHARED((_NACC, _H), jnp.float32),
        pltpu.SemaphoreType.DMA,
        pltpu.SemaphoreType.DMA,
        pltpu.SemaphoreType.DMA,
    ],
)
def _agg_kernel(hn_hbm, src_hbm, dst_hbm, out_hbm,
                src_v, dst_v, rows_v, zb_v, acc_sh, sem_g, sem_s0, sem_s1):
    cid = lax.axis_index("c")
    sid = lax.axis_index("s")

    def zstep(i, _):
        zb_v[i] = jnp.zeros((_H,), jnp.float32)
        return 0

    lax.fori_loop(0, _ZR, zstep, 0)
    pltpu.sync_copy(zb_v, acc_sh.at[pl.ds(sid * _ZR, _ZR)])
    plsc.subcore_barrier()

    nrows = lax.select(cid == 0, _RBIG, _RSMALL)
    base = pl.multiple_of(
        lax.select(cid == 0, _NS * _RSMALL + sid * _RBIG, sid * _RSMALL), 8)
    pltpu.sync_copy(src_hbm.at[pl.ds(base, _RBIG)], src_v)
    pltpu.sync_copy(dst_hbm.at[pl.ds(base, _RBIG)], dst_v)

    nblk = nrows // 16

    for t in range(8):
        pltpu.async_copy(hn_hbm.at[src_v.at[t]], rows_v.at[t], sem_g)

    def blk(k, _):
        j0 = k * 16

        @pl.when(k >= 1)
        def _():
            for t in range(8):
                pltpu.make_async_copy(rows_v.at[8 + t],
                                      acc_sh.at[dst_v.at[j0 - 8 + t]],
                                      sem_s1).wait()

        for t in range(8):
            pltpu.async_copy(hn_hbm.at[src_v.at[j0 + 8 + t]],
                             rows_v.at[8 + t], sem_g)
        for t in range(8):
            pltpu.make_async_copy(hn_hbm.at[src_v.at[j0 + t]],
                                  rows_v.at[t], sem_g).wait()
            pltpu.async_copy(rows_v.at[t], acc_sh.at[dst_v.at[j0 + t]],
                             sem_s0, add=True)
        for t in range(8):
            pltpu.make_async_copy(rows_v.at[t], acc_sh.at[dst_v.at[j0 + t]],
                                  sem_s0).wait()
        for t in range(8):
            pltpu.make_async_copy(hn_hbm.at[src_v.at[j0 + 8 + t]],
                                  rows_v.at[8 + t], sem_g).wait()
            pltpu.async_copy(rows_v.at[8 + t], acc_sh.at[dst_v.at[j0 + 8 + t]],
                             sem_s1, add=True)

        @pl.when(k + 1 < nblk)
        def _():
            for t in range(8):
                pltpu.async_copy(hn_hbm.at[src_v.at[j0 + 16 + t]],
                                 rows_v.at[t], sem_g)
        return 0

    lax.fori_loop(0, nblk, blk, 0)
    for t in range(8):
        pltpu.make_async_copy(rows_v.at[8 + t], acc_sh.at[dst_v.at[0]],
                              sem_s1).wait()
    plsc.subcore_barrier()
    pltpu.sync_copy(acc_sh.at[pl.ds(sid * _ZR, _ZR)],
                    out_hbm.at[cid, pl.ds(sid * _ZR, _ZR)])


def _dense1_body(dgp, xw, w1b, hnp, dvp):
    d = dgp[...]
    dinvp = lax.rsqrt(d[0] + d[1] + 1.0)
    dvp[...] = dinvp
    hnp[...] = jnp.dot(xw[...], w1b[...],
                       preferred_element_type=jnp.float32) * dinvp


def _dense2_body(ap, hnp, dvp, b1p, w2b, hn2p):
    p = ap[...]
    dinvp = dvp[...]
    sp = jnp.maximum((p[0] + p[1] + hnp[...]) * dinvp + b1p[...], 0.0)
    hn2p[...] = jnp.dot(sp, w2b[...], preferred_element_type=jnp.float32) * dinvp


def _dense3_body(ap, hnp, dvp, b2p, wob, bop, outp):
    p = ap[...]
    dinvp = dvp[...]
    sp = jnp.maximum((p[0] + p[1] + hnp[...]) * dinvp + b2p[...], 0.0)
    lp = jnp.dot(sp, wob[...], preferred_element_type=jnp.float32) + bop[...]
    outs = []
    for k in range(8):
        lk = lp[:, 40 * k:40 * (k + 1)]
        m = jnp.max(lk, axis=1, keepdims=True)
        lse = jnp.log(jnp.sum(jnp.exp(lk - m), axis=1, keepdims=True)) + m
        outs.append(lk - lse)
    outp[...] = jnp.concatenate(outs, axis=1)[:_N // 8, :]


def kernel(x, edge_index, W1, b1, W2, b2, Wo, bo):
    src = edge_index[0]
    dst = edge_index[1]
    pad = _EP - _E
    fill = jnp.arange(pad, dtype=jnp.int32)
    srcp = jnp.concatenate([src, fill % _N]).reshape(_ROWS, _CH)
    dump = _N + (fill % (_NACC - _N))
    dstp = jnp.concatenate([dst, dump]).reshape(_ROWS, _CH)

    degrep = _deg_kernel(dstp)

    _P8 = _NACC // 8
    xw = jnp.reshape(jnp.pad(x, ((0, _NACC - _N), (0, 0))), (_P8, 8 * _D))
    w1b = block_diag(*([W1] * 8))
    hn1p, dinvp = pl.pallas_call(
        _dense1_body,
        out_shape=[jax.ShapeDtypeStruct((_P8, 128), jnp.float32),
                   jax.ShapeDtypeStruct((_P8, 128), jnp.float32)],
    )(jnp.reshape(degrep, (_NC, _P8, 128)), xw, w1b)

    w2b = block_diag(*([W2] * 8))
    b1p = jnp.tile(b1, 8).reshape(1, 128)

    a1 = _agg_kernel(jnp.reshape(hn1p, (_NACC, _H)), srcp, dstp)
    hn2p = pl.pallas_call(
        _dense2_body,
        out_shape=jax.ShapeDtypeStruct((_P8, 128), jnp.float32),
    )(jnp.reshape(a1, (_NC, _P8, 128)), hn1p, dinvp, b1p, w2b)

    wob = block_diag(*([Wo] * 8))
    b2p = jnp.tile(b2, 8).reshape(1, 128)
    bop = jnp.tile(bo, 8).reshape(1, 8 * _C)

    a2 = _agg_kernel(jnp.reshape(hn2p, (_NACC, _H)), srcp, dstp)
    outp = pl.pallas_call(
        _dense3_body,
        out_shape=jax.ShapeDtypeStruct((_N // 8, 8 * _C), jnp.float32),
    )(jnp.reshape(a2, (_NC, _P8, 128)), hn2p, dinvp, b2p, wob, bop)
    return jnp.reshape(outp, (_N, _C))
